# baseline (speedup 1.0000x reference)
import jax
import jax.numpy as jnp
from jax.experimental import pallas as pl
from jax.experimental.pallas import tpu as pltpu

ALPHA = 0.2
NEG = -9e15


def _proj1_kernel(x_ref, W_ref, A1_ref, A2_ref, Wh16_ref, el_ref, er_ref):
    xb = x_ref[...]
    nheads = W_ref.shape[0]
    el_cols, er_cols = [], []
    for h in range(nheads):
        Wh = jnp.dot(xb, W_ref[h], preferred_element_type=jnp.float32)
        Wh16_ref[h] = Wh.astype(jnp.bfloat16)
        el_cols.append(jnp.dot(Wh, A1_ref[h], preferred_element_type=jnp.float32))
        er_cols.append(jnp.dot(Wh, A2_ref[h], preferred_element_type=jnp.float32))
    el_ref[...] = jnp.concatenate(el_cols, axis=1)
    er_ref[...] = jnp.concatenate(er_cols, axis=1)


def _masked_softmax_matmul(adjf, el_col, er_row, W16, nrows):
    e = el_col + er_row
    e = jnp.maximum(e, ALPHA * e)
    m = jnp.max(e, axis=1, keepdims=True)
    p = jnp.exp(e - m) * adjf
    s = jnp.sum(p, axis=1, keepdims=True)
    hp = jnp.dot(p.astype(jnp.bfloat16), W16,
                 preferred_element_type=jnp.float32)
    inv = jnp.where(s > 0, 1.0 / s, 0.0)
    colmean = jnp.sum(W16, axis=0, dtype=jnp.float32,
                      keepdims=True) * (1.0 / nrows)
    fb = jnp.where(s > 0, 0.0, 1.0)
    return hp * inv + fb * colmean


def _attn1_kernel(adj_ref, Wh16_ref, el_ref, erT_ref, Wo16_ref, a1o_ref,
                  a2o_ref, Whout16_ref, el2_ref, er2_ref):
    nheads = Wh16_ref.shape[0]
    n = Wh16_ref.shape[1]
    adjf = (adj_ref[...] > 0).astype(jnp.float32)
    acc = None
    for h in range(nheads):
        hp = _masked_softmax_matmul(adjf, el_ref[:, h:h + 1],
                                    erT_ref[h][None, :], Wh16_ref[h], n)
        hp = jnp.where(hp > 0, hp, jnp.exp(hp) - 1.0)
        part = jnp.dot(hp.astype(jnp.bfloat16), Wo16_ref[h],
                       preferred_element_type=jnp.float32)
        acc = part if acc is None else acc + part
    Whout16_ref[...] = acc.astype(jnp.bfloat16)
    el2_ref[...] = jnp.dot(acc, a1o_ref[...], preferred_element_type=jnp.float32)
    er2_ref[...] = jnp.dot(acc, a2o_ref[...], preferred_element_type=jnp.float32)


def _attn2_kernel(adj_ref, Whout16_ref, el2_ref, er2T_ref, out_ref):
    n = Whout16_ref.shape[0]
    adjf = (adj_ref[...] > 0).astype(jnp.float32)
    out_ref[...] = _masked_softmax_matmul(adjf, el2_ref[...],
                                          er2T_ref[...], Whout16_ref[...], n)


def kernel(x, adj, W0, a0, W1, a1, W2, a2, W3, a3, W_out, a_out):
    n, nfeat = x.shape
    nhid = W0.shape[1]
    nheads = 4
    nclass = W_out.shape[1]
    ncp = 128 * ((nclass + 127) // 128)
    br = min(256, n)
    nblk = n // br

    Ws = jnp.stack([W0, W1, W2, W3])
    A1 = jnp.stack([a0[:nhid], a1[:nhid], a2[:nhid], a3[:nhid]])
    A2 = jnp.stack([a0[nhid:], a1[nhid:], a2[nhid:], a3[nhid:]])
    Wo16 = jnp.zeros((nheads, nhid, ncp), jnp.float32).at[:, :, :nclass].set(
        W_out.reshape(nheads, nhid, nclass)).astype(jnp.bfloat16)
    a1o = jnp.zeros((ncp, 1), jnp.float32).at[:nclass].set(a_out[:nclass])
    a2o = jnp.zeros((ncp, 1), jnp.float32).at[:nclass].set(a_out[nclass:])

    full = lambda shape: pl.BlockSpec(shape, lambda i: (0,) * len(shape))
    params = pltpu.CompilerParams(dimension_semantics=("parallel",))

    Wh16, el, er = pl.pallas_call(
        _proj1_kernel,
        grid=(nblk,),
        in_specs=[
            pl.BlockSpec((br, nfeat), lambda i: (i, 0)),
            full(Ws.shape),
            full(A1.shape),
            full(A2.shape),
        ],
        out_specs=[
            pl.BlockSpec((nheads, br, nhid), lambda i: (0, i, 0)),
            pl.BlockSpec((br, nheads), lambda i: (i, 0)),
            pl.BlockSpec((br, nheads), lambda i: (i, 0)),
        ],
        out_shape=[
            jax.ShapeDtypeStruct((nheads, n, nhid), jnp.bfloat16),
            jax.ShapeDtypeStruct((n, nheads), jnp.float32),
            jax.ShapeDtypeStruct((n, nheads), jnp.float32),
        ],
        compiler_params=params,
    )(x, Ws, A1, A2)

    erT = er.T

    Whout16, el2, er2 = pl.pallas_call(
        _attn1_kernel,
        grid=(nblk,),
        in_specs=[
            pl.BlockSpec((br, n), lambda i: (i, 0)),
            full(Wh16.shape),
            pl.BlockSpec((br, nheads), lambda i: (i, 0)),
            full(erT.shape),
            full(Wo16.shape),
            full(a1o.shape),
            full(a2o.shape),
        ],
        out_specs=[
            pl.BlockSpec((br, ncp), lambda i: (i, 0)),
            pl.BlockSpec((br, 1), lambda i: (i, 0)),
            pl.BlockSpec((br, 1), lambda i: (i, 0)),
        ],
        out_shape=[
            jax.ShapeDtypeStruct((n, ncp), jnp.bfloat16),
            jax.ShapeDtypeStruct((n, 1), jnp.float32),
            jax.ShapeDtypeStruct((n, 1), jnp.float32),
        ],
        compiler_params=params,
    )(adj, Wh16, el, erT, Wo16, a1o, a2o)

    er2T = er2.reshape(1, n)

    out = pl.pallas_call(
        _attn2_kernel,
        grid=(nblk,),
        in_specs=[
            pl.BlockSpec((br, n), lambda i: (i, 0)),
            full(Whout16.shape),
            pl.BlockSpec((br, 1), lambda i: (i, 0)),
            full(er2T.shape),
        ],
        out_specs=pl.BlockSpec((br, ncp), lambda i: (i, 0)),
        out_shape=jax.ShapeDtypeStruct((n, ncp), jnp.float32),
        compiler_params=params,
    )(adj, Whout16, el2, er2T)

    return out[:, :nclass]


# speedup vs baseline: 1.3418x; 1.3418x over previous
import jax
import jax.numpy as jnp
from jax.experimental import pallas as pl
from jax.experimental.pallas import tpu as pltpu

ALPHA = 0.2


def _proj1_kernel(x_ref, W_ref, A1_ref, A2_ref, Wh16_ref, el_ref, er_ref,
                  csum_ref, ermax_ref):
    xb = x_ref[...]
    nheads = W_ref.shape[0]
    el_cols, er_cols, cs_rows, em_rows = [], [], [], []
    for h in range(nheads):
        Wh = jnp.dot(xb, W_ref[h], preferred_element_type=jnp.float32)
        Wh16_ref[h] = Wh.astype(jnp.bfloat16)
        el_cols.append(jnp.dot(Wh, A1_ref[h], preferred_element_type=jnp.float32))
        er_h = jnp.dot(Wh, A2_ref[h], preferred_element_type=jnp.float32)
        er_cols.append(er_h)
        cs_rows.append(jnp.sum(Wh, axis=0, keepdims=True))
        em_rows.append(jnp.max(er_h))
    el_ref[...] = jnp.concatenate(el_cols, axis=1)
    er_ref[...] = jnp.concatenate(er_cols, axis=1)
    csum_ref[...] = jnp.stack(cs_rows)[None]
    ermax_ref[...] = jnp.stack(em_rows).reshape(1, 1, nheads)


def _leaky(z):
    return jnp.maximum(z, ALPHA * z)


def _sep_softmax_matmul(adjb, el_col, er_row, ermax, W16, colmean):
    z = el_col + ermax
    m = _leaky(z)
    a = jnp.exp(z - m).astype(jnp.bfloat16)
    c = jnp.exp(ALPHA * z - m).astype(jnp.bfloat16)
    dr = er_row - ermax
    b = jnp.exp(dr).astype(jnp.bfloat16)
    d = jnp.exp(ALPHA * dr).astype(jnp.bfloat16)
    p = jnp.maximum(a * b, c * d) * adjb
    s = jnp.sum(p, axis=1, keepdims=True, dtype=jnp.float32)
    hp = jnp.dot(p, W16, preferred_element_type=jnp.float32)
    inv = jnp.where(s > 0, 1.0 / s, 0.0)
    fb = jnp.where(s > 0, 0.0, 1.0)
    return hp * inv + fb * colmean


def _attn1_kernel(adj_ref, Wh16_ref, el_ref, erT_ref, ermax_ref, cm1_ref,
                  Wo16_ref, a1o_ref, a2o_ref, Whout16_ref, el2_ref, er2_ref,
                  csum2_ref, er2max_ref):
    nheads = Wh16_ref.shape[0]
    adjb = (adj_ref[...] > 0).astype(jnp.bfloat16)
    acc = None
    for h in range(nheads):
        hp = _sep_softmax_matmul(adjb, el_ref[:, h:h + 1],
                                 erT_ref[h][None, :], ermax_ref[:, h:h + 1],
                                 Wh16_ref[h], cm1_ref[h:h + 1, :])
        hp = jnp.where(hp > 0, hp, jnp.exp(hp) - 1.0)
        part = jnp.dot(hp.astype(jnp.bfloat16), Wo16_ref[h],
                       preferred_element_type=jnp.float32)
        acc = part if acc is None else acc + part
    Whout16_ref[...] = acc.astype(jnp.bfloat16)
    er2 = jnp.dot(acc, a2o_ref[...], preferred_element_type=jnp.float32)
    el2_ref[...] = jnp.dot(acc, a1o_ref[...], preferred_element_type=jnp.float32)
    er2_ref[...] = er2
    csum2_ref[...] = jnp.sum(acc, axis=0, keepdims=True)[None]
    er2max_ref[...] = jnp.max(er2).reshape(1, 1, 1)


def _attn2_kernel(adj_ref, Whout16_ref, el2_ref, er2T_ref, er2max_ref,
                  cm2_ref, out_ref):
    adjb = (adj_ref[...] > 0).astype(jnp.bfloat16)
    out_ref[...] = _sep_softmax_matmul(adjb, el2_ref[...], er2T_ref[...],
                                       er2max_ref[...], Whout16_ref[...],
                                       cm2_ref[...])


def kernel(x, adj, W0, a0, W1, a1, W2, a2, W3, a3, W_out, a_out):
    n, nfeat = x.shape
    nhid = W0.shape[1]
    nheads = 4
    nclass = W_out.shape[1]
    ncp = 128 * ((nclass + 127) // 128)
    br = min(256, n)
    nblk = n // br

    Ws = jnp.stack([W0, W1, W2, W3])
    A1 = jnp.stack([a0[:nhid], a1[:nhid], a2[:nhid], a3[:nhid]])
    A2 = jnp.stack([a0[nhid:], a1[nhid:], a2[nhid:], a3[nhid:]])
    Wo16 = jnp.zeros((nheads, nhid, ncp), jnp.float32).at[:, :, :nclass].set(
        W_out.reshape(nheads, nhid, nclass)).astype(jnp.bfloat16)
    a1o = jnp.zeros((ncp, 1), jnp.float32).at[:nclass].set(a_out[:nclass])
    a2o = jnp.zeros((ncp, 1), jnp.float32).at[:nclass].set(a_out[nclass:])

    full = lambda shape: pl.BlockSpec(shape, lambda i: (0,) * len(shape))
    params = pltpu.CompilerParams(dimension_semantics=("parallel",))

    Wh16, el, er, csum, ermaxb = pl.pallas_call(
        _proj1_kernel,
        grid=(nblk,),
        in_specs=[
            pl.BlockSpec((br, nfeat), lambda i: (i, 0)),
            full(Ws.shape),
            full(A1.shape),
            full(A2.shape),
        ],
        out_specs=[
            pl.BlockSpec((nheads, br, nhid), lambda i: (0, i, 0)),
            pl.BlockSpec((br, nheads), lambda i: (i, 0)),
            pl.BlockSpec((br, nheads), lambda i: (i, 0)),
            pl.BlockSpec((1, nheads, 1, nhid), lambda i: (i, 0, 0, 0)),
            pl.BlockSpec((1, 1, nheads), lambda i: (i, 0, 0)),
        ],
        out_shape=[
            jax.ShapeDtypeStruct((nheads, n, nhid), jnp.bfloat16),
            jax.ShapeDtypeStruct((n, nheads), jnp.float32),
            jax.ShapeDtypeStruct((n, nheads), jnp.float32),
            jax.ShapeDtypeStruct((nblk, nheads, 1, nhid), jnp.float32),
            jax.ShapeDtypeStruct((nblk, 1, nheads), jnp.float32),
        ],
        compiler_params=params,
    )(x, Ws, A1, A2)

    erT = er.T
    ermax = jnp.max(ermaxb, axis=0)
    cm1 = jnp.sum(csum, axis=0)[:, 0, :] * (1.0 / n)

    Whout16, el2, er2, csum2, er2maxb = pl.pallas_call(
        _attn1_kernel,
        grid=(nblk,),
        in_specs=[
            pl.BlockSpec((br, n), lambda i: (i, 0)),
            full(Wh16.shape),
            pl.BlockSpec((br, nheads), lambda i: (i, 0)),
            full(erT.shape),
            full(ermax.shape),
            full(cm1.shape),
            full(Wo16.shape),
            full(a1o.shape),
            full(a2o.shape),
        ],
        out_specs=[
            pl.BlockSpec((br, ncp), lambda i: (i, 0)),
            pl.BlockSpec((br, 1), lambda i: (i, 0)),
            pl.BlockSpec((br, 1), lambda i: (i, 0)),
            pl.BlockSpec((1, 1, ncp), lambda i: (i, 0, 0)),
            pl.BlockSpec((1, 1, 1), lambda i: (i, 0, 0)),
        ],
        out_shape=[
            jax.ShapeDtypeStruct((n, ncp), jnp.bfloat16),
            jax.ShapeDtypeStruct((n, 1), jnp.float32),
            jax.ShapeDtypeStruct((n, 1), jnp.float32),
            jax.ShapeDtypeStruct((nblk, 1, ncp), jnp.float32),
            jax.ShapeDtypeStruct((nblk, 1, 1), jnp.float32),
        ],
        compiler_params=params,
    )(adj, Wh16, el, erT, ermax, cm1, Wo16, a1o, a2o)

    er2T = er2.reshape(1, n)
    er2max = jnp.max(er2maxb, axis=0)
    cm2 = jnp.sum(csum2, axis=0) * (1.0 / n)

    out = pl.pallas_call(
        _attn2_kernel,
        grid=(nblk,),
        in_specs=[
            pl.BlockSpec((br, n), lambda i: (i, 0)),
            full(Whout16.shape),
            pl.BlockSpec((br, 1), lambda i: (i, 0)),
            full(er2T.shape),
            full(er2max.shape),
            full(cm2.shape),
        ],
        out_specs=pl.BlockSpec((br, ncp), lambda i: (i, 0)),
        out_shape=jax.ShapeDtypeStruct((n, ncp), jnp.float32),
        compiler_params=params,
    )(adj, Whout16, el2, er2T, er2max, cm2)

    return out[:, :nclass]
